# Initial kernel scaffold; baseline (speedup 1.0000x reference)
import jax
import jax.numpy as jnp
from jax.experimental import pallas as pl


def kernel(logits, label):
    raise NotImplementedError("write your pallas kernel here")


# speedup vs baseline: 4.0848x; 4.0848x over previous
import jax
import jax.numpy as jnp
from jax.experimental import pallas as pl
from jax.experimental.pallas import tpu as pltpu

LB_SMOOTH_ = 0.1
IGNORE_INDEX_ = 255
H_BLOCK = 128


def _ce_kernel(x_ref, lab_ref, loss_ref, cnt_ref):
    h = pl.program_id(1)

    x = x_ref[0]
    lab = lab_ref[0]
    num_classes = x.shape[0]

    m = jnp.max(x, axis=0)
    s = jnp.sum(jnp.exp(x - m[None]), axis=0)
    lse = m + jnp.log(s)
    sum_x = jnp.sum(x, axis=0)

    ignore = lab == IGNORE_INDEX_
    lab_c = jnp.where(ignore, 0, lab)
    cls = jax.lax.broadcasted_iota(jnp.int32, x.shape, 0)
    x_tgt = jnp.sum(jnp.where(cls == lab_c[None], x, 0.0), axis=0)

    lb_pos = 1.0 - LB_SMOOTH_
    lb_neg = LB_SMOOTH_ / num_classes
    lp_tgt = x_tgt - lse
    sum_logs = sum_x - num_classes * lse
    loss = -((lb_pos - lb_neg) * lp_tgt + lb_neg * sum_logs)
    loss = jnp.where(ignore, 0.0, loss)

    part = jnp.sum(loss).reshape(1, 1, 1)
    cnt = jnp.sum((~ignore).astype(jnp.float32)).reshape(1, 1, 1)

    @pl.when(h == 0)
    def _init():
        loss_ref[...] = part
        cnt_ref[...] = cnt

    @pl.when(h != 0)
    def _acc():
        loss_ref[...] += part
        cnt_ref[...] += cnt


def kernel(logits, label):
    n, c, hh, w = logits.shape
    label = label.astype(jnp.int32)
    grid = (n, hh // H_BLOCK)

    loss_sums, cnts = pl.pallas_call(
        _ce_kernel,
        grid=grid,
        in_specs=[
            pl.BlockSpec((1, c, H_BLOCK, w), lambda i, j: (i, 0, j, 0)),
            pl.BlockSpec((1, H_BLOCK, w), lambda i, j: (i, j, 0)),
        ],
        out_specs=[
            pl.BlockSpec((1, 1, 1), lambda i, j: (i, 0, 0)),
            pl.BlockSpec((1, 1, 1), lambda i, j: (i, 0, 0)),
        ],
        out_shape=[
            jax.ShapeDtypeStruct((n, 1, 1), jnp.float32),
            jax.ShapeDtypeStruct((n, 1, 1), jnp.float32),
        ],
        compiler_params=pltpu.CompilerParams(
            dimension_semantics=("parallel", "arbitrary"),
        ),
    )(logits.astype(jnp.float32), label)

    return jnp.sum(loss_sums) / jnp.sum(cnts)
